# Initial kernel scaffold; baseline (speedup 1.0000x reference)
import functools

import jax
import jax.numpy as jnp
from jax import lax
from jax.experimental import pallas as pl
from jax.experimental.pallas import tpu as pltpu
from jax.experimental.pallas import tpu_sc as plsc

_N = 10000
_E = 320000
_D = 128

_NC = 2
_NS = 16
_CHUNK = 128
_G = 80
_E_PAD = _NC * _NS * _G * _CHUNK
_N_PAD = 10240
_RPT = _N_PAD // _NS
_BN = 512
_NB = _N_PAD // _BN

_mesh = plsc.VectorSubcoreMesh(core_axis_name="c", subcore_axis_name="s")


@functools.partial(
    pl.kernel,
    out_type=jax.ShapeDtypeStruct((_NC, _N_PAD, 16), jnp.float32),
    mesh=_mesh,
    scratch_types=[
        pltpu.VMEM((_CHUNK,), jnp.int32),
        pltpu.VMEM((_CHUNK, 16), jnp.float32),
        pltpu.VMEM_SHARED((_N_PAD, 16), jnp.float32),
    ],
)
def _deg_kernel(col_hbm, ones_hbm, zeros_hbm, out_hbm, idx_v, ones_v, acc):
    cid = lax.axis_index("c")
    sid = lax.axis_index("s")
    pltpu.sync_copy(ones_hbm, ones_v)
    pltpu.sync_copy(zeros_hbm.at[pl.ds(sid * _RPT, _RPT)],
                    acc.at[pl.ds(sid * _RPT, _RPT)])
    plsc.subcore_barrier()
    tile_base = (cid * _NS + sid) * _G

    def body(g, carry):
        pltpu.sync_copy(col_hbm.at[tile_base + g], idx_v)
        pltpu.sync_copy(ones_v, acc.at[idx_v], add=True)
        return carry

    lax.fori_loop(0, _G, body, 0)
    plsc.subcore_barrier()
    pltpu.sync_copy(acc.at[pl.ds(sid * _RPT, _RPT)],
                    out_hbm.at[cid, pl.ds(sid * _RPT, _RPT)])


@functools.partial(
    pl.kernel,
    out_type=jax.ShapeDtypeStruct((_NC, _N_PAD, _D), jnp.float32),
    mesh=_mesh,
    scratch_types=[
        pltpu.VMEM((_CHUNK,), jnp.int32),
        pltpu.VMEM((_CHUNK,), jnp.int32),
        pltpu.VMEM((_CHUNK, _D), jnp.float32),
        pltpu.VMEM_SHARED((_N_PAD, _D), jnp.float32),
        pltpu.SemaphoreType.DMA,
    ],
)
def _edge_kernel(h_hbm, row_hbm, col_hbm, zeros_hbm, out_hbm,
                 idxr_v, idxc_v, rows_v, acc, sem):
    cid = lax.axis_index("c")
    sid = lax.axis_index("s")
    pltpu.sync_copy(zeros_hbm.at[pl.ds(sid * _RPT, _RPT)],
                    acc.at[pl.ds(sid * _RPT, _RPT)])
    plsc.subcore_barrier()
    tile_base = (cid * _NS + sid) * _G

    def body(g, carry):
        pltpu.sync_copy(row_hbm.at[tile_base + g], idxr_v)
        pltpu.async_copy(h_hbm.at[idxr_v], rows_v, sem).wait()
        pltpu.sync_copy(col_hbm.at[tile_base + g], idxc_v)
        pltpu.sync_copy(rows_v, acc.at[idxc_v], add=True)
        return carry

    lax.fori_loop(0, _G, body, 0)
    plsc.subcore_barrier()
    pltpu.sync_copy(acc.at[pl.ds(sid * _RPT, _RPT)],
                    out_hbm.at[cid, pl.ds(sid * _RPT, _RPT)])


def _tck1_body(degp_ref, x_ref, w1_ref, h1p_ref, dinv_ref):
    i = pl.program_id(0)
    dega = degp_ref[0, pl.ds(i * _BN, _BN), :]
    degb = degp_ref[1, pl.ds(i * _BN, _BN), :]
    deg = dega[:, :1] + degb[:, :1] + 1.0
    dinv = lax.rsqrt(deg)
    dinvb = jnp.broadcast_to(dinv, (_BN, _D))
    dinv_ref[...] = dinvb
    h = jnp.dot(x_ref[...], w1_ref[...], preferred_element_type=jnp.float32)
    h1p_ref[...] = dinvb * h


def _tck1(degp, x, w1):
    return pl.pallas_call(
        _tck1_body,
        grid=(_NB,),
        in_specs=[
            pl.BlockSpec((_NC, _N_PAD, 16), lambda i: (0, 0, 0)),
            pl.BlockSpec((_BN, _D), lambda i: (i, 0)),
            pl.BlockSpec((_D, _D), lambda i: (0, 0)),
        ],
        out_specs=[
            pl.BlockSpec((_BN, _D), lambda i: (i, 0)),
            pl.BlockSpec((_BN, _D), lambda i: (i, 0)),
        ],
        out_shape=[
            jax.ShapeDtypeStruct((_N, _D), jnp.float32),
            jax.ShapeDtypeStruct((_N, _D), jnp.float32),
        ],
    )(degp, x, w1)


def _tck2_body(s1p_ref, h1p_ref, dinv_ref, b1_ref, w2_ref, h2p_ref):
    s = s1p_ref[0] + s1p_ref[1]
    o1 = jnp.maximum(dinv_ref[...] * (s + h1p_ref[...]) + b1_ref[...], 0.0)
    h2 = jnp.dot(o1, w2_ref[...], preferred_element_type=jnp.float32)
    h2p_ref[...] = dinv_ref[...] * h2


def _tck2(s1p, h1p, dinvb, b1, w2):
    return pl.pallas_call(
        _tck2_body,
        grid=(_NB,),
        in_specs=[
            pl.BlockSpec((_NC, _BN, _D), lambda i: (0, i, 0)),
            pl.BlockSpec((_BN, _D), lambda i: (i, 0)),
            pl.BlockSpec((_BN, _D), lambda i: (i, 0)),
            pl.BlockSpec((1, _D), lambda i: (0, 0)),
            pl.BlockSpec((_D, _D), lambda i: (0, 0)),
        ],
        out_specs=pl.BlockSpec((_BN, _D), lambda i: (i, 0)),
        out_shape=jax.ShapeDtypeStruct((_N, _D), jnp.float32),
    )(s1p, h1p, dinvb, b1, w2)


def _tck3_body(s2p_ref, h2p_ref, dinv_ref, b2_ref, out_ref):
    s = s2p_ref[0] + s2p_ref[1]
    out_ref[...] = dinv_ref[...] * (s + h2p_ref[...]) + b2_ref[...]


def _tck3(s2p, h2p, dinvb, b2):
    return pl.pallas_call(
        _tck3_body,
        grid=(_NB,),
        in_specs=[
            pl.BlockSpec((_NC, _BN, _D), lambda i: (0, i, 0)),
            pl.BlockSpec((_BN, _D), lambda i: (i, 0)),
            pl.BlockSpec((_BN, _D), lambda i: (i, 0)),
            pl.BlockSpec((1, _D), lambda i: (0, 0)),
        ],
        out_specs=pl.BlockSpec((_BN, _D), lambda i: (i, 0)),
        out_shape=jax.ShapeDtypeStruct((_N, _D), jnp.float32),
    )(s2p, h2p, dinvb, b2)


def kernel(x, edge_index, W1, b1, W2, b2):
    row = edge_index[0]
    col = edge_index[1]
    pad = _E_PAD - _E
    ar = jnp.arange(pad, dtype=jnp.int32)
    row_p = jnp.concatenate([row, ar % _N])
    col_p = jnp.concatenate([col, _N + (ar % 128)])
    row2d = row_p.reshape(_E_PAD // _CHUNK, _CHUNK)
    col2d = col_p.reshape(_E_PAD // _CHUNK, _CHUNK)

    zeros_d = jnp.zeros((_N_PAD, _D), jnp.float32)
    zeros_16 = jnp.zeros((_N_PAD, 16), jnp.float32)
    ones_16 = jnp.ones((_CHUNK, 16), jnp.float32)

    degp = _deg_kernel(col2d, ones_16, zeros_16)
    h1p, dinvb = _tck1(degp, x, W1)
    s1p = _edge_kernel(h1p, row2d, col2d, zeros_d)
    h2p = _tck2(s1p, h1p, dinvb, b1.reshape(1, _D), W2)
    s2p = _edge_kernel(h2p, row2d, col2d, zeros_d)
    out = _tck3(s2p, h2p, dinvb, b2.reshape(1, _D))
    return out


# speedup vs baseline: 14.7541x; 14.7541x over previous
import functools

import jax
import jax.numpy as jnp
from jax import lax
from jax.experimental import pallas as pl
from jax.experimental.pallas import tpu as pltpu
from jax.experimental.pallas import tpu_sc as plsc

_N = 10000
_E = 320000
_D = 128

_NC = 2
_NS = 16
_CHUNK = 128
_G = 80
_E_PAD = _NC * _NS * _G * _CHUNK
_N_PAD = 10240
_RPT = _N_PAD // _NS
_DW = 128
_BN = 512
_NB = _N_PAD // _BN

_mesh = plsc.VectorSubcoreMesh(core_axis_name="c", subcore_axis_name="s",
                               num_cores=_NC, num_subcores=_NS)


def _deg_body(col_hbm, ones_hbm, zeros_hbm, out_hbm, idx_v, ones_v, acc):
    cid = lax.axis_index("c")
    sid = lax.axis_index("s")
    pltpu.sync_copy(ones_hbm, ones_v)
    pltpu.sync_copy(zeros_hbm.at[pl.ds(sid * _RPT, _RPT)],
                    acc.at[pl.ds(sid * _RPT, _RPT)])
    plsc.subcore_barrier()
    tile_base = (cid * _NS + sid) * _G

    def body(g, carry):
        pltpu.sync_copy(col_hbm.at[tile_base + g], idx_v)
        pltpu.sync_copy(ones_v, acc.at[idx_v], add=True)
        return carry

    lax.fori_loop(0, _G, body, 0)
    plsc.subcore_barrier()
    pltpu.sync_copy(acc.at[pl.ds(sid * _RPT, _RPT)],
                    out_hbm.at[cid, pl.ds(sid * _RPT, _RPT)])


def _make_deg_kernel(interpret=False):
    return functools.partial(
        pl.kernel,
        out_type=jax.ShapeDtypeStruct((_NC, _N_PAD, _DW), jnp.float32),
        mesh=_mesh,
        interpret=interpret,
        scratch_types=[
            pltpu.VMEM((_CHUNK,), jnp.int32),
            pltpu.VMEM((_CHUNK, _DW), jnp.float32),
            pltpu.VMEM_SHARED((_N_PAD, _DW), jnp.float32),
        ],
    )(_deg_body)


_deg_kernel = _make_deg_kernel()


def _edge_body(h_hbm, row_hbm, col_hbm, zeros_hbm, out_hbm,
               idxr_v, idxc_v, rows_v, acc, sem):
    cid = lax.axis_index("c")
    sid = lax.axis_index("s")
    pltpu.sync_copy(zeros_hbm.at[pl.ds(sid * _RPT, _RPT)],
                    acc.at[pl.ds(sid * _RPT, _RPT)])
    plsc.subcore_barrier()
    tile_base = (cid * _NS + sid) * _G

    def body(g, carry):
        pltpu.sync_copy(row_hbm.at[tile_base + g], idxr_v)
        pltpu.async_copy(h_hbm.at[idxr_v], rows_v, sem).wait()
        pltpu.sync_copy(col_hbm.at[tile_base + g], idxc_v)
        pltpu.sync_copy(rows_v, acc.at[idxc_v], add=True)
        return carry

    lax.fori_loop(0, _G, body, 0)
    plsc.subcore_barrier()
    pltpu.sync_copy(acc.at[pl.ds(sid * _RPT, _RPT)],
                    out_hbm.at[cid, pl.ds(sid * _RPT, _RPT)])


def _make_edge_kernel(interpret=False):
    return functools.partial(
        pl.kernel,
        out_type=jax.ShapeDtypeStruct((_NC, _N_PAD, _D), jnp.float32),
        mesh=_mesh,
        interpret=interpret,
        scratch_types=[
            pltpu.VMEM((_CHUNK,), jnp.int32),
            pltpu.VMEM((_CHUNK,), jnp.int32),
            pltpu.VMEM((_CHUNK, _D), jnp.float32),
            pltpu.VMEM_SHARED((_N_PAD, _D), jnp.float32),
            pltpu.SemaphoreType.DMA,
        ],
    )(_edge_body)


_edge_kernel = _make_edge_kernel()


def _tck1_body(degp_ref, x_ref, w1_ref, h1p_ref, dinv_ref):
    i = pl.program_id(0)
    dega = degp_ref[0, pl.ds(i * _BN, _BN), :]
    degb = degp_ref[1, pl.ds(i * _BN, _BN), :]
    deg = dega[:, :1] + degb[:, :1] + 1.0
    dinv = lax.rsqrt(deg)
    dinvb = jnp.broadcast_to(dinv, (_BN, _D))
    dinv_ref[...] = dinvb
    h = jnp.dot(x_ref[...], w1_ref[...], preferred_element_type=jnp.float32)
    h1p_ref[...] = dinvb * h


def _tck1(degp, x, w1):
    return pl.pallas_call(
        _tck1_body,
        grid=(_NB,),
        in_specs=[
            pl.BlockSpec((_NC, _N_PAD, _DW), lambda i: (0, 0, 0)),
            pl.BlockSpec((_BN, _D), lambda i: (i, 0)),
            pl.BlockSpec((_D, _D), lambda i: (0, 0)),
        ],
        out_specs=[
            pl.BlockSpec((_BN, _D), lambda i: (i, 0)),
            pl.BlockSpec((_BN, _D), lambda i: (i, 0)),
        ],
        out_shape=[
            jax.ShapeDtypeStruct((_N, _D), jnp.float32),
            jax.ShapeDtypeStruct((_N, _D), jnp.float32),
        ],
    )(degp, x, w1)


def _tck2_body(s1p_ref, h1p_ref, dinv_ref, b1_ref, w2_ref, h2p_ref):
    s = s1p_ref[0] + s1p_ref[1]
    o1 = jnp.maximum(dinv_ref[...] * (s + h1p_ref[...]) + b1_ref[...], 0.0)
    h2 = jnp.dot(o1, w2_ref[...], preferred_element_type=jnp.float32)
    h2p_ref[...] = dinv_ref[...] * h2


def _tck2(s1p, h1p, dinvb, b1, w2):
    return pl.pallas_call(
        _tck2_body,
        grid=(_NB,),
        in_specs=[
            pl.BlockSpec((_NC, _BN, _D), lambda i: (0, i, 0)),
            pl.BlockSpec((_BN, _D), lambda i: (i, 0)),
            pl.BlockSpec((_BN, _D), lambda i: (i, 0)),
            pl.BlockSpec((1, _D), lambda i: (0, 0)),
            pl.BlockSpec((_D, _D), lambda i: (0, 0)),
        ],
        out_specs=pl.BlockSpec((_BN, _D), lambda i: (i, 0)),
        out_shape=jax.ShapeDtypeStruct((_N, _D), jnp.float32),
    )(s1p, h1p, dinvb, b1, w2)


def _tck3_body(s2p_ref, h2p_ref, dinv_ref, b2_ref, out_ref):
    s = s2p_ref[0] + s2p_ref[1]
    out_ref[...] = dinv_ref[...] * (s + h2p_ref[...]) + b2_ref[...]


def _tck3(s2p, h2p, dinvb, b2):
    return pl.pallas_call(
        _tck3_body,
        grid=(_NB,),
        in_specs=[
            pl.BlockSpec((_NC, _BN, _D), lambda i: (0, i, 0)),
            pl.BlockSpec((_BN, _D), lambda i: (i, 0)),
            pl.BlockSpec((_BN, _D), lambda i: (i, 0)),
            pl.BlockSpec((1, _D), lambda i: (0, 0)),
        ],
        out_specs=pl.BlockSpec((_BN, _D), lambda i: (i, 0)),
        out_shape=jax.ShapeDtypeStruct((_N, _D), jnp.float32),
    )(s2p, h2p, dinvb, b2)


def kernel(x, edge_index, W1, b1, W2, b2):
    row = edge_index[0]
    col = edge_index[1]
    pad = _E_PAD - _E
    ar = jnp.arange(pad, dtype=jnp.int32)
    row_p = jnp.concatenate([row, ar % _N])
    col_p = jnp.concatenate([col, _N + (ar % 128)])
    row2d = row_p.reshape(_E_PAD // _CHUNK, _CHUNK)
    col2d = col_p.reshape(_E_PAD // _CHUNK, _CHUNK)

    zeros_d = jnp.zeros((_N_PAD, _D), jnp.float32)
    zeros_w = jnp.zeros((_N_PAD, _DW), jnp.float32)
    ones_w = jnp.ones((_CHUNK, _DW), jnp.float32)

    degp = _deg_kernel(col2d, ones_w, zeros_w)
    h1p, dinvb = _tck1(degp, x, W1)
    s1p = _edge_kernel(h1p, row2d, col2d, zeros_d)
    h2p = _tck2(s1p, h1p, dinvb, b1.reshape(1, _D), W2)
    s2p = _edge_kernel(h2p, row2d, col2d, zeros_d)
    out = _tck3(s2p, h2p, dinvb, b2.reshape(1, _D))
    return out


# speedup vs baseline: 23.6449x; 1.6026x over previous
import functools

import jax
import jax.numpy as jnp
from jax import lax
from jax.experimental import pallas as pl
from jax.experimental.pallas import tpu as pltpu
from jax.experimental.pallas import tpu_sc as plsc

_N = 10000
_E = 320000
_D = 128

_NC = 2
_NS = 16
_CHUNK = 128
_G = 80
_E_PAD = _NC * _NS * _G * _CHUNK
_N_PAD = 10240
_RPT = _N_PAD // _NS
_DW = 64
_BN = 512
_NB = _N_PAD // _BN

_mesh = plsc.VectorSubcoreMesh(core_axis_name="c", subcore_axis_name="s",
                               num_cores=_NC, num_subcores=_NS)


def _deg_body(col_hbm, ones_hbm, zeros_hbm, out_hbm, idxc_all, ones_v, sem, acc):
    cid = lax.axis_index("c")
    sid = lax.axis_index("s")
    pltpu.sync_copy(ones_hbm, ones_v)
    pltpu.sync_copy(zeros_hbm.at[pl.ds(sid * _RPT, _RPT)],
                    acc.at[pl.ds(sid * _RPT, _RPT)])
    tile_base = (cid * _NS + sid) * _G
    pltpu.sync_copy(col_hbm.at[pl.ds(tile_base, _G)], idxc_all)
    plsc.subcore_barrier()

    def body(g, carry):
        pltpu.async_copy(ones_v, acc.at[idxc_all.at[g]], sem, add=True)
        return carry

    lax.fori_loop(0, _G, body, 0)

    def drain(g, carry):
        pltpu.make_async_copy(ones_v, acc.at[idxc_all.at[0]], sem).wait()
        return carry

    lax.fori_loop(0, _G, drain, 0)
    plsc.subcore_barrier()
    pltpu.sync_copy(acc.at[pl.ds(sid * _RPT, _RPT)],
                    out_hbm.at[cid, pl.ds(sid * _RPT, _RPT)])


def _make_deg_kernel(interpret=False):
    return functools.partial(
        pl.kernel,
        out_type=jax.ShapeDtypeStruct((_NC, _N_PAD, _DW), jnp.float32),
        mesh=_mesh,
        interpret=interpret,
        scratch_types=[
            pltpu.VMEM((_G, _CHUNK), jnp.int32),
            pltpu.VMEM((_CHUNK, _DW), jnp.float32),
            pltpu.SemaphoreType.DMA,
            pltpu.VMEM_SHARED((_N_PAD, _DW), jnp.float32),
        ],
    )(_deg_body)


_deg_kernel = _make_deg_kernel()


_EC = 128
_EG = (_E_PAD // (_NC * _NS)) // _EC
_NBUF = 2
_NSEG = 2
_GSEG = _EG // _NSEG


def _edge_body(h_hbm, row_hbm, col_hbm, zeros_hbm, out_hbm,
               idxr_h, idxc_h, rows, gsem, ssem, acc):
    cid = lax.axis_index("c")
    sid = lax.axis_index("s")
    pltpu.sync_copy(zeros_hbm.at[pl.ds(sid * _RPT, _RPT)],
                    acc.at[pl.ds(sid * _RPT, _RPT)])
    tile_base = (cid * _NS + sid) * _EG
    plsc.subcore_barrier()

    nq = _GSEG // _NBUF
    for seg in range(_NSEG):
        if seg > 0:
            for b in range(_NBUF):
                pltpu.make_async_copy(rows.at[b], acc.at[idxc_h.at[0]],
                                      ssem[b]).wait()
        base = tile_base + seg * _GSEG
        pltpu.sync_copy(row_hbm.at[pl.ds(base, _GSEG)], idxr_h)
        pltpu.sync_copy(col_hbm.at[pl.ds(base, _GSEG)], idxc_h)

        def body(q, carry):
            descs = []
            for b in range(_NBUF):
                c = q * _NBUF + b

                @pl.when(q >= 1)
                def _():
                    pltpu.make_async_copy(
                        rows.at[b], acc.at[idxc_h.at[c]], ssem[b]).wait()

                descs.append(pltpu.async_copy(
                    h_hbm.at[idxr_h.at[c]], rows.at[b], gsem[b]))
            for b in range(_NBUF):
                c = q * _NBUF + b
                descs[b].wait()
                pltpu.async_copy(rows.at[b], acc.at[idxc_h.at[c]], ssem[b],
                                 add=True)
            return carry

        lax.fori_loop(0, nq, body, 0)
    for b in range(_NBUF):
        pltpu.make_async_copy(rows.at[b], acc.at[idxc_h.at[b]],
                              ssem[b]).wait()
    plsc.subcore_barrier()
    pltpu.sync_copy(acc.at[pl.ds(sid * _RPT, _RPT)],
                    out_hbm.at[cid, pl.ds(sid * _RPT, _RPT)])


def _make_edge_kernel(interpret=False):
    return functools.partial(
        pl.kernel,
        out_type=jax.ShapeDtypeStruct((_NC, _N_PAD, _D), jnp.float32),
        mesh=_mesh,
        interpret=interpret,
        scratch_types=[
            pltpu.VMEM((_GSEG, _EC), jnp.int32),
            pltpu.VMEM((_GSEG, _EC), jnp.int32),
            pltpu.VMEM((_NBUF, _EC, _D), jnp.float32),
            [pltpu.SemaphoreType.DMA] * _NBUF,
            [pltpu.SemaphoreType.DMA] * _NBUF,
            pltpu.VMEM_SHARED((_N_PAD, _D), jnp.float32),
        ],
    )(_edge_body)


_edge_kernel = _make_edge_kernel()


def _tck0_body(x_ref, w1_ref, u1_ref):
    u1_ref[...] = jnp.dot(x_ref[...], w1_ref[...],
                          preferred_element_type=jnp.float32)


def _tck0(x, w1):
    return pl.pallas_call(
        _tck0_body,
        grid=(_NB,),
        in_specs=[
            pl.BlockSpec((_BN, _D), lambda i: (i, 0)),
            pl.BlockSpec((_D, _D), lambda i: (0, 0)),
        ],
        out_specs=pl.BlockSpec((_BN, _D), lambda i: (i, 0)),
        out_shape=jax.ShapeDtypeStruct((_N, _D), jnp.float32),
    )(x, w1)


def _tck1_body(degp_ref, u1_ref, h1p_ref, dinv_ref):
    i = pl.program_id(0)
    dega = degp_ref[0, pl.ds(i * _BN, _BN), :]
    degb = degp_ref[1, pl.ds(i * _BN, _BN), :]
    deg = dega[:, :1] + degb[:, :1] + 1.0
    dinv = lax.rsqrt(deg)
    dinvb = jnp.broadcast_to(dinv, (_BN, _D))
    dinv_ref[...] = dinvb
    h1p_ref[...] = dinvb * u1_ref[...]


def _tck1(degp, u1):
    return pl.pallas_call(
        _tck1_body,
        grid=(_NB,),
        in_specs=[
            pl.BlockSpec((_NC, _N_PAD, _DW), lambda i: (0, 0, 0)),
            pl.BlockSpec((_BN, _D), lambda i: (i, 0)),
        ],
        out_specs=[
            pl.BlockSpec((_BN, _D), lambda i: (i, 0)),
            pl.BlockSpec((_BN, _D), lambda i: (i, 0)),
        ],
        out_shape=[
            jax.ShapeDtypeStruct((_N, _D), jnp.float32),
            jax.ShapeDtypeStruct((_N, _D), jnp.float32),
        ],
    )(degp, u1)


def _tck2_body(s1p_ref, h1p_ref, dinv_ref, b1_ref, w2_ref, h2p_ref):
    s = s1p_ref[0] + s1p_ref[1]
    o1 = jnp.maximum(dinv_ref[...] * (s + h1p_ref[...]) + b1_ref[...], 0.0)
    h2 = jnp.dot(o1, w2_ref[...], preferred_element_type=jnp.float32)
    h2p_ref[...] = dinv_ref[...] * h2


def _tck2(s1p, h1p, dinvb, b1, w2):
    return pl.pallas_call(
        _tck2_body,
        grid=(_NB,),
        in_specs=[
            pl.BlockSpec((_NC, _BN, _D), lambda i: (0, i, 0)),
            pl.BlockSpec((_BN, _D), lambda i: (i, 0)),
            pl.BlockSpec((_BN, _D), lambda i: (i, 0)),
            pl.BlockSpec((1, _D), lambda i: (0, 0)),
            pl.BlockSpec((_D, _D), lambda i: (0, 0)),
        ],
        out_specs=pl.BlockSpec((_BN, _D), lambda i: (i, 0)),
        out_shape=jax.ShapeDtypeStruct((_N, _D), jnp.float32),
    )(s1p, h1p, dinvb, b1, w2)


def _tck3_body(s2p_ref, h2p_ref, dinv_ref, b2_ref, out_ref):
    s = s2p_ref[0] + s2p_ref[1]
    out_ref[...] = dinv_ref[...] * (s + h2p_ref[...]) + b2_ref[...]


def _tck3(s2p, h2p, dinvb, b2):
    return pl.pallas_call(
        _tck3_body,
        grid=(_NB,),
        in_specs=[
            pl.BlockSpec((_NC, _BN, _D), lambda i: (0, i, 0)),
            pl.BlockSpec((_BN, _D), lambda i: (i, 0)),
            pl.BlockSpec((_BN, _D), lambda i: (i, 0)),
            pl.BlockSpec((1, _D), lambda i: (0, 0)),
        ],
        out_specs=pl.BlockSpec((_BN, _D), lambda i: (i, 0)),
        out_shape=jax.ShapeDtypeStruct((_N, _D), jnp.float32),
    )(s2p, h2p, dinvb, b2)


def kernel(x, edge_index, W1, b1, W2, b2):
    row = edge_index[0]
    col = edge_index[1]
    pad = _E_PAD - _E
    ar = jnp.arange(pad, dtype=jnp.int32)
    row_p = jnp.concatenate([row, ar % _N])
    col_p = jnp.concatenate([col, _N + (ar % 128)])
    row2d_e = row_p.reshape(_E_PAD // _EC, _EC)
    col2d_e = col_p.reshape(_E_PAD // _EC, _EC)
    col2d_d = col_p.reshape(_E_PAD // _CHUNK, _CHUNK)

    zeros_d = jnp.zeros((_N_PAD, _D), jnp.float32)
    zeros_w = jnp.zeros((_N_PAD, _DW), jnp.float32)
    ones_w = jnp.ones((_CHUNK, _DW), jnp.float32)

    u1 = _tck0(x, W1)
    degp = _deg_kernel(col2d_d, ones_w, zeros_w)
    h1p, dinvb = _tck1(degp, u1)
    s1p = _edge_kernel(h1p, row2d_e, col2d_e, zeros_d)
    h2p = _tck2(s1p, h1p, dinvb, b1.reshape(1, _D), W2)
    s2p = _edge_kernel(h2p, row2d_e, col2d_e, zeros_d)
    out = _tck3(s2p, h2p, dinvb, b2.reshape(1, _D))
    return out
